# baseline (speedup 1.0000x reference)
import jax
import jax.numpy as jnp
from jax.experimental import pallas as pl
from jax.experimental.pallas import tpu as pltpu

_BS = 1024


def _router_kernel(x_ref, w_ref, lo_ref, oh_ref, pm_ref,
                   lo_s, oh_s, pm_s, sems):
    i = pl.program_id(0)
    n = pl.num_programs(0)
    sb = x_ref.shape[1]
    spb = lo_ref.shape[1] // sb
    p = i % 2
    bi = i // spb
    so = (i % spb) * sb

    logits = jax.lax.dot_general(
        x_ref[0], w_ref[...], (((1,), (1,)), ((), ())),
        preferred_element_type=jnp.float32)
    rmax = jnp.max(logits, axis=1, keepdims=True)
    ssum = jnp.sum(jnp.exp(logits - rmax), axis=1, keepdims=True)
    e = logits.shape[1]
    iota = jax.lax.broadcasted_iota(jnp.int32, logits.shape, 1)
    idx = jnp.min(jnp.where(logits == rmax, iota, e), axis=1, keepdims=True)

    @pl.when(i >= 2)
    def _():
        pltpu.make_async_copy(lo_s.at[p], lo_s.at[p], sems.at[0, p]).wait()
        pltpu.make_async_copy(oh_s.at[p], oh_s.at[p], sems.at[1, p]).wait()
        pltpu.make_async_copy(pm_s.at[p], pm_s.at[p], sems.at[2, p]).wait()

    lo_s[p] = logits
    oh_s[p] = (iota == idx).astype(jnp.int32)
    pm_s[p] = 1.0 / ssum

    row = pl.ds(so, sb)
    lo_cp = pltpu.make_async_copy(lo_s.at[p], lo_ref.at[bi, row, :],
                                  sems.at[0, p])
    oh_cp = pltpu.make_async_copy(oh_s.at[p], oh_ref.at[bi, row, :],
                                  sems.at[1, p])
    pm_cp = pltpu.make_async_copy(pm_s.at[p], pm_ref.at[bi, row, :],
                                  sems.at[2, p])
    lo_cp.start()
    oh_cp.start()
    pm_cp.start()

    @pl.when(i == n - 1)
    def _():
        q = (i + 1) % 2
        pltpu.make_async_copy(lo_s.at[q], lo_s.at[q], sems.at[0, q]).wait()
        pltpu.make_async_copy(oh_s.at[q], oh_s.at[q], sems.at[1, q]).wait()
        pltpu.make_async_copy(pm_s.at[q], pm_s.at[q], sems.at[2, q]).wait()
        lo_cp.wait()
        oh_cp.wait()
        pm_cp.wait()


def kernel(hidden_states, W):
    b, s, h = hidden_states.shape
    e = W.shape[0]
    nsteps = (b * s) // _BS

    logits, onehot, pmax = pl.pallas_call(
        _router_kernel,
        grid=(nsteps,),
        in_specs=[
            pl.BlockSpec((1, _BS, h),
                         lambda i, _spb=s // _BS: (i // _spb, i % _spb, 0)),
            pl.BlockSpec((e, h), lambda i: (0, 0)),
        ],
        out_specs=[
            pl.BlockSpec(memory_space=pl.ANY),
            pl.BlockSpec(memory_space=pl.ANY),
            pl.BlockSpec(memory_space=pl.ANY),
        ],
        out_shape=[
            jax.ShapeDtypeStruct((b, s, e), jnp.float32),
            jax.ShapeDtypeStruct((b, s, e), jnp.int32),
            jax.ShapeDtypeStruct((b, s, 1), jnp.float32),
        ],
        scratch_shapes=[
            pltpu.VMEM((2, _BS, e), jnp.float32),
            pltpu.VMEM((2, _BS, e), jnp.int32),
            pltpu.VMEM((2, _BS, 1), jnp.float32),
            pltpu.SemaphoreType.DMA((3, 2)),
        ],
    )(hidden_states, W)

    return (onehot, pmax, logits)


# speedup vs baseline: 1.0012x; 1.0012x over previous
import jax
import jax.numpy as jnp
from jax.experimental import pallas as pl

_BS = 1024


def _router_kernel(x_ref, w_ref, logits_ref, onehot_ref, pmax_ref):
    logits = jax.lax.dot_general(
        x_ref[0], w_ref[...], (((1,), (1,)), ((), ())),
        preferred_element_type=jnp.float32)
    rmax = jnp.max(logits, axis=1, keepdims=True)
    ssum = jnp.sum(jnp.exp(logits - rmax), axis=1, keepdims=True)
    pmax_ref[0] = 1.0 / ssum
    e = logits.shape[1]
    iota = jax.lax.broadcasted_iota(jnp.int32, logits.shape, 1)
    idx = jnp.min(jnp.where(logits == rmax, iota, e), axis=1, keepdims=True)
    onehot_ref[0] = (iota == idx).astype(jnp.int32)
    logits_ref[0] = logits


def kernel(hidden_states, W):
    b, s, h = hidden_states.shape
    e = W.shape[0]

    logits, onehot, pmax = pl.pallas_call(
        _router_kernel,
        grid=(b, s // _BS),
        in_specs=[
            pl.BlockSpec((1, _BS, h), lambda i, j: (i, j, 0)),
            pl.BlockSpec((e, h), lambda i, j: (0, 0)),
        ],
        out_specs=[
            pl.BlockSpec((1, _BS, e), lambda i, j: (i, j, 0)),
            pl.BlockSpec((1, _BS, e), lambda i, j: (i, j, 0)),
            pl.BlockSpec((1, _BS, 1), lambda i, j: (i, j, 0)),
        ],
        out_shape=[
            jax.ShapeDtypeStruct((b, s, e), jnp.float32),
            jax.ShapeDtypeStruct((b, s, e), jnp.int32),
            jax.ShapeDtypeStruct((b, s, 1), jnp.float32),
        ],
    )(hidden_states, W)

    return (onehot, pmax, logits)


# speedup vs baseline: 1.0722x; 1.0710x over previous
import jax
import jax.numpy as jnp
from jax.experimental import pallas as pl

_BS = 1024


def _router_kernel(x_ref, w_ref, logits_ref, onehot_ref, pmax_ref):
    logits = jax.lax.dot_general(
        x_ref[0], w_ref[...], (((1,), (1,)), ((), ())),
        preferred_element_type=jnp.float32)
    rmax = jnp.max(logits, axis=1, keepdims=True)
    ssum = jnp.sum(jnp.exp(logits - rmax), axis=1, keepdims=True)
    pm = 1.0 / ssum
    sb = pm.shape[0]
    rows = [jnp.swapaxes(jax.lax.slice(pm, (r * 128, 0), ((r + 1) * 128, 1)),
                         0, 1) for r in range(sb // 128)]
    pmax_ref[...] = jax.lax.concatenate(rows, 0)
    e = logits.shape[1]
    iota = jax.lax.broadcasted_iota(jnp.int32, logits.shape, 1)
    idx = jnp.min(jnp.where(logits == rmax, iota, e), axis=1, keepdims=True)
    onehot_ref[0] = (iota == idx).astype(jnp.int32)
    logits_ref[0] = logits


def kernel(hidden_states, W):
    b, s, h = hidden_states.shape
    e = W.shape[0]

    logits, onehot, pmax = pl.pallas_call(
        _router_kernel,
        grid=(b, s // _BS),
        in_specs=[
            pl.BlockSpec((1, _BS, h), lambda i, j: (i, j, 0)),
            pl.BlockSpec((e, h), lambda i, j: (0, 0)),
        ],
        out_specs=[
            pl.BlockSpec((1, _BS, e), lambda i, j: (i, j, 0)),
            pl.BlockSpec((1, _BS, e), lambda i, j: (i, j, 0)),
            pl.BlockSpec((None, None, _BS // 128, 128),
                         lambda i, j: (i, j, 0, 0)),
        ],
        out_shape=[
            jax.ShapeDtypeStruct((b, s, e), jnp.float32),
            jax.ShapeDtypeStruct((b, s, e), jnp.int32),
            jax.ShapeDtypeStruct((b, s // _BS, _BS // 128, 128), jnp.float32),
        ],
    )(hidden_states, W)

    return (onehot, pmax.reshape(b, s, 1), logits)
